# baseline (speedup 1.0000x reference)
import jax
import jax.numpy as jnp
from jax.experimental import pallas as pl
from jax.experimental.pallas import tpu as pltpu

T_BLK = 2048
D_BLK = 2048
E = 64
K = 8
ND = 2


def _router_kernel(x_ref, w_ref, bias_ref, out_ref, acc_ref):
    j = pl.program_id(1)
    partial = jnp.dot(x_ref[...], w_ref[...], preferred_element_type=jnp.float32)

    @pl.when(j == 0)
    def _():
        acc_ref[...] = partial

    @pl.when(j > 0)
    def _():
        acc_ref[...] += partial

    @pl.when(j == ND - 1)
    def _():
        logits = acc_ref[...] + bias_ref[0:1, :]
        cur = logits
        m0 = None
        for _ in range(K):
            m = jnp.max(cur, axis=1, keepdims=True)
            if m0 is None:
                m0 = m
            cur = jnp.where(cur == m, -jnp.inf, cur)
        ex = jnp.where(cur < logits, jnp.exp(logits - m0), 0.0)
        z = jnp.sum(ex, axis=1, keepdims=True)
        out_ref[...] = ex / z


@jax.jit
def kernel(x, w_gate, b_gate, expert_biases):
    T, D = x.shape
    bias = jnp.broadcast_to((b_gate + expert_biases)[None, :], (8, E))
    grid = (T // T_BLK, ND)
    return pl.pallas_call(
        _router_kernel,
        grid=grid,
        in_specs=[
            pl.BlockSpec((T_BLK, D_BLK), lambda i, j: (i, j)),
            pl.BlockSpec((D_BLK, E), lambda i, j: (j, 0)),
            pl.BlockSpec((8, E), lambda i, j: (0, 0)),
        ],
        out_specs=pl.BlockSpec((T_BLK, E), lambda i, j: (i, 0)),
        out_shape=jax.ShapeDtypeStruct((T, E), x.dtype),
        scratch_shapes=[pltpu.VMEM((T_BLK, E), jnp.float32)],
        compiler_params=pltpu.CompilerParams(
            dimension_semantics=("parallel", "arbitrary"),
        ),
    )(x, w_gate, bias)


# speedup vs baseline: 1.1363x; 1.1363x over previous
import jax
import jax.numpy as jnp
from jax.experimental import pallas as pl
from jax.experimental.pallas import tpu as pltpu

T_BLK = 1024
E = 64
K = 8


def _router_kernel(x_ref, w_ref, bias_ref, out_ref):
    logits = jnp.dot(x_ref[...], w_ref[...], preferred_element_type=jnp.float32)
    logits = logits + bias_ref[0:1, :]

    cur = logits
    m0 = None
    for _ in range(K):
        m = jnp.max(cur, axis=1, keepdims=True)
        if m0 is None:
            m0 = m
        cur = jnp.where(cur == m, -jnp.inf, cur)

    ex = jnp.where(cur < logits, jnp.exp(logits - m0), 0.0)
    z = jnp.sum(ex, axis=1, keepdims=True)
    out_ref[...] = ex / z


@jax.jit
def kernel(x, w_gate, b_gate, expert_biases):
    T, D = x.shape
    bias = jnp.broadcast_to((b_gate + expert_biases)[None, :], (8, E))
    grid = (T // T_BLK,)
    return pl.pallas_call(
        _router_kernel,
        grid=grid,
        in_specs=[
            pl.BlockSpec((T_BLK, D), lambda i: (i, 0)),
            pl.BlockSpec((D, E), lambda i: (0, 0)),
            pl.BlockSpec((8, E), lambda i: (0, 0)),
        ],
        out_specs=pl.BlockSpec((T_BLK, E), lambda i: (i, 0)),
        out_shape=jax.ShapeDtypeStruct((T, E), x.dtype),
        compiler_params=pltpu.CompilerParams(
            dimension_semantics=("parallel",),
        ),
    )(x, w_gate, bias)
